# Initial kernel scaffold; baseline (speedup 1.0000x reference)
import jax
import jax.numpy as jnp
from jax.experimental import pallas as pl


def kernel(unit_type, edge_index, edge_attr, node2graph, input, emb, mlp_w1, mlp_b1, mlp_w2, mlp_b2, gru_w_ih, gru_w_hh, gru_b_ih, gru_b_hh, lstm_w_ih, lstm_w_hh, lstm_b_ih, lstm_b_hh):
    raise NotImplementedError("write your pallas kernel here")


# speedup vs baseline: 1.6752x; 1.6752x over previous
import functools

import jax
import jax.numpy as jnp
from jax import lax
from jax.experimental import pallas as pl
from jax.experimental.pallas import tpu as pltpu
from jax.experimental.pallas import tpu_sc as plsc

N = 10000
E = 160000
H = 32
ED = 16
NUM_UNIT = 256
B = 128
NP = 10240
CH = 128
NCHUNKS = E // CH
NSUB = 16
NCORE = 2
ZR = NP // NSUB


def _emb_body(ut_ref, emb_ref, out_ref):
    ut = ut_ref[...]
    oh = (ut == lax.broadcasted_iota(jnp.int32, (ut.shape[0], NUM_UNIT), 1))
    out_ref[...] = jnp.dot(oh.astype(jnp.float32), emb_ref[...],
                           preferred_element_type=jnp.float32)


def _embed(ut_col, emb):
    TA = 2560
    return pl.pallas_call(
        _emb_body,
        grid=(NP // TA,),
        in_specs=[
            pl.BlockSpec((TA, 1), lambda i: (i, 0)),
            pl.BlockSpec((NUM_UNIT, H), lambda i: (0, 0)),
        ],
        out_specs=pl.BlockSpec((TA, H), lambda i: (i, 0)),
        out_shape=jax.ShapeDtypeStruct((NP, H), jnp.float32),
    )(ut_col, emb)


def _gather_sc(table, nin2d):
    mesh = plsc.VectorSubcoreMesh(core_axis_name="c", subcore_axis_name="s")
    cpw = (NCHUNKS + NCORE * NSUB - 1) // (NCORE * NSUB)

    @functools.partial(
        pl.kernel, mesh=mesh,
        out_type=jax.ShapeDtypeStruct((E, H), jnp.float32),
        compiler_params=pltpu.CompilerParams(use_tc_tiling_on_sc=False),
        scratch_types=[
            pltpu.VMEM((CH,), jnp.int32),
            pltpu.VMEM((CH, H), jnp.float32),
            pltpu.SemaphoreType.DMA,
        ],
    )
    def k(table_hbm, nin_hbm, out_hbm, idx_v, rows_v, sem):
        wid = lax.axis_index("c") * NSUB + lax.axis_index("s")

        def body(j, carry):
            chunk = wid + j * (NCORE * NSUB)

            @pl.when(chunk < NCHUNKS)
            def _():
                pltpu.sync_copy(nin_hbm.at[chunk], idx_v)
                pltpu.async_copy(table_hbm.at[idx_v], rows_v, sem).wait()
                pltpu.sync_copy(rows_v, out_hbm.at[pl.ds(chunk * CH, CH)])

            return carry

        lax.fori_loop(0, cpw, body, 0)

    return k(table, nin2d)


def _edge_body(ea_ref, xs_ref, w1_ref, b1_ref, a_ref, b2_ref, out_ref):
    ea = ea_ref[...]
    xs = xs_ref[...]
    hmid = jnp.maximum(
        jnp.dot(ea, w1_ref[...], preferred_element_type=jnp.float32)
        + b1_ref[0:1, :], 0.0)
    y = jnp.dot(xs, a_ref[...], preferred_element_type=jnp.float32)
    acc = jnp.dot(xs, b2_ref[...], preferred_element_type=jnp.float32)
    for k in range(H):
        acc = acc + hmid[:, k:k + 1] * y[:, k * H:(k + 1) * H]
    out_ref[...] = acc


def _edge_msg(edge_attr, x_src, w1, b1r, a_mat, b2_mat):
    TE = 2000
    return pl.pallas_call(
        _edge_body,
        grid=(E // TE,),
        in_specs=[
            pl.BlockSpec((TE, ED), lambda i: (i, 0)),
            pl.BlockSpec((TE, H), lambda i: (i, 0)),
            pl.BlockSpec((ED, H), lambda i: (0, 0)),
            pl.BlockSpec((8, H), lambda i: (0, 0)),
            pl.BlockSpec((H, H * H), lambda i: (0, 0)),
            pl.BlockSpec((H, H), lambda i: (0, 0)),
        ],
        out_specs=pl.BlockSpec((TE, H), lambda i: (i, 0)),
        out_shape=jax.ShapeDtypeStruct((E, H), jnp.float32),
    )(edge_attr, x_src, w1, b1r, a_mat, b2_mat)


def _scatter_sc(msg, nout2d, zeros_np):
    mesh = plsc.VectorSubcoreMesh(core_axis_name="c", subcore_axis_name="s")
    half = NCHUNKS // NCORE
    cpw = (half + NSUB - 1) // NSUB

    @functools.partial(
        pl.kernel, mesh=mesh,
        out_type=jax.ShapeDtypeStruct((NCORE, NP, H), jnp.float32),
        compiler_params=pltpu.CompilerParams(use_tc_tiling_on_sc=False),
        scratch_types=[
            pltpu.VMEM((CH,), jnp.int32),
            pltpu.VMEM((CH, H), jnp.float32),
            pltpu.VMEM_SHARED((NP, H), jnp.float32),
        ],
    )
    def k(msg_hbm, nout_hbm, zeros_hbm, parts_hbm, idx_v, rows_v, acc_sh):
        c = lax.axis_index("c")
        s = lax.axis_index("s")
        pltpu.sync_copy(zeros_hbm.at[pl.ds(s * ZR, ZR)],
                        acc_sh.at[pl.ds(s * ZR, ZR)])
        plsc.subcore_barrier()

        def body(j, carry):
            chunk = c * half + s + j * NSUB

            @pl.when(chunk < (c + 1) * half)
            def _():
                pltpu.sync_copy(nout_hbm.at[chunk], idx_v)
                pltpu.sync_copy(msg_hbm.at[pl.ds(chunk * CH, CH)], rows_v)
                pltpu.sync_copy(rows_v, acc_sh.at[idx_v], add=True)

            return carry

        lax.fori_loop(0, cpw, body, 0)
        plsc.subcore_barrier()
        pltpu.sync_copy(acc_sh.at[pl.ds(s * ZR, ZR)],
                        parts_hbm.at[c, pl.ds(s * ZR, ZR)])

    return k(msg, nout2d, zeros_np)


def _gru_body(up_ref, li_ref, giT_ref, ghT_ref, gb_ref, nf_ref):
    x = jnp.maximum(up_ref[0] + up_ref[1], 0.0)
    hx = li_ref[...]
    gi = jnp.dot(x, giT_ref[...], preferred_element_type=jnp.float32) \
        + gb_ref[0:1, :]
    gh = jnp.dot(hx, ghT_ref[...], preferred_element_type=jnp.float32) \
        + gb_ref[1:2, :]
    r = jax.nn.sigmoid(gi[:, :H] + gh[:, :H])
    z = jax.nn.sigmoid(gi[:, H:2 * H] + gh[:, H:2 * H])
    nn_ = jnp.tanh(gi[:, 2 * H:] + r * gh[:, 2 * H:])
    nf_ref[...] = (1.0 - z) * nn_ + z * hx


def _gru(parts, li, giT, ghT, gb):
    TN = 2560
    return pl.pallas_call(
        _gru_body,
        grid=(NP // TN,),
        in_specs=[
            pl.BlockSpec((2, TN, H), lambda i: (0, i, 0)),
            pl.BlockSpec((TN, H), lambda i: (i, 0)),
            pl.BlockSpec((H, 3 * H), lambda i: (0, 0)),
            pl.BlockSpec((H, 3 * H), lambda i: (0, 0)),
            pl.BlockSpec((8, 3 * H), lambda i: (0, 0)),
        ],
        out_specs=pl.BlockSpec((TN, H), lambda i: (i, 0)),
        out_shape=jax.ShapeDtypeStruct((NP, H), jnp.float32),
    )(parts, li, giT, ghT, gb)


def _s2s_body(nf_ref, nfT_ref, n2gr_ref, liT_ref, lhT_ref, lb_ref, qs_ref):
    nf = nf_ref[...]
    nfT = nfT_ref[...]
    n2gr = n2gr_ref[0:1, :]
    ohT = (n2gr == lax.broadcasted_iota(jnp.int32, (B, NP), 0))
    ohf = ohT.astype(jnp.float32)

    h_l = jnp.zeros((B, H), jnp.float32)
    c_l = jnp.zeros((B, H), jnp.float32)
    q_star = jnp.zeros((B, 2 * H), jnp.float32)
    for _ in range(3):
        gates = jnp.dot(q_star, liT_ref[...], preferred_element_type=jnp.float32) \
            + jnp.dot(h_l, lhT_ref[...], preferred_element_type=jnp.float32) \
            + lb_ref[0:1, :]
        ig = jax.nn.sigmoid(gates[:, :H])
        fg = jax.nn.sigmoid(gates[:, H:2 * H])
        gg = jnp.tanh(gates[:, 2 * H:3 * H])
        og = jax.nn.sigmoid(gates[:, 3 * H:])
        c_l = fg * c_l + ig * gg
        h_l = og * jnp.tanh(c_l)
        q = h_l
        s = jnp.dot(q, nfT, preferred_element_type=jnp.float32)
        product = jnp.sum(ohf * s, axis=0, keepdims=True)
        pmax_b = jnp.max(jnp.where(ohT, product, -1e30), axis=1,
                         keepdims=True)
        pmax_n = jnp.sum(jnp.where(ohT, pmax_b, 0.0), axis=0,
                         keepdims=True)
        pexp = jnp.exp(product - pmax_n)
        psum_b = jnp.sum(ohf * pexp, axis=1, keepdims=True)
        psum_n = jnp.sum(jnp.where(ohT, psum_b, 0.0), axis=0,
                         keepdims=True)
        att = pexp / (psum_n + 1e-10)
        out = jnp.dot(ohf * att, nf, preferred_element_type=jnp.float32)
        q_star = jnp.concatenate([q, out], axis=1)
    qs_ref[...] = q_star


def _s2s(nf, nfT, n2gr, liT, lhT, lb):
    return pl.pallas_call(
        _s2s_body,
        out_shape=jax.ShapeDtypeStruct((B, 2 * H), jnp.float32),
        compiler_params=pltpu.CompilerParams(
            vmem_limit_bytes=64 * 1024 * 1024),
    )(nf, nfT, n2gr, liT, lhT, lb)


def kernel(unit_type, edge_index, edge_attr, node2graph, input, emb,
           mlp_w1, mlp_b1, mlp_w2, mlp_b2,
           gru_w_ih, gru_w_hh, gru_b_ih, gru_b_hh,
           lstm_w_ih, lstm_w_hh, lstm_b_ih, lstm_b_hh):
    ut_col = jnp.concatenate(
        [unit_type.astype(jnp.int32), jnp.zeros((NP - N,), jnp.int32)]
    ).reshape(NP, 1)
    n2g = jnp.concatenate(
        [node2graph.astype(jnp.int32), jnp.full((NP - N,), B, jnp.int32)])
    n2gr = jnp.broadcast_to(n2g.reshape(1, NP), (8, NP))

    nin2d = edge_index[0].astype(jnp.int32).reshape(NCHUNKS, CH)
    nout2d = edge_index[1].astype(jnp.int32).reshape(NCHUNKS, CH)

    a_mat = mlp_w2.reshape(H, H, H).transpose(2, 0, 1).reshape(H, H * H)
    b2_mat = mlp_b2.reshape(H, H).T
    b1r = jnp.broadcast_to(mlp_b1.reshape(1, H), (8, H))

    giT = gru_w_ih.T
    ghT = gru_w_hh.T
    gb = jnp.concatenate(
        [gru_b_ih.reshape(1, 3 * H), gru_b_hh.reshape(1, 3 * H)], axis=0)
    gb = jnp.concatenate([gb, jnp.zeros((6, 3 * H), jnp.float32)], axis=0)
    liT = lstm_w_ih.T
    lhT = lstm_w_hh.T
    lb = jnp.broadcast_to(
        (lstm_b_ih + lstm_b_hh).reshape(1, 4 * H), (8, 4 * H))

    layer_input = _embed(ut_col, emb)
    x_src = _gather_sc(layer_input, nin2d)
    msg = _edge_msg(edge_attr, x_src, mlp_w1, b1r, a_mat, b2_mat)
    parts = _scatter_sc(msg, nout2d, jnp.zeros((NP, H), jnp.float32))
    nf = _gru(parts, layer_input, giT, ghT, gb)
    q_star = _s2s(nf, nf.T, n2gr, liT, lhT, lb)
    return (q_star, nf[:N])


# speedup vs baseline: 5.5419x; 3.3082x over previous
import functools

import jax
import jax.numpy as jnp
from jax import lax
from jax.experimental import pallas as pl
from jax.experimental.pallas import tpu as pltpu
from jax.experimental.pallas import tpu_sc as plsc

N = 10000
E = 160000
H = 32
ED = 16
NUM_UNIT = 256
B = 128
NP = 10240
CH = 128
NCHUNKS = E // CH
NSUB = 16
NCORE = 2
ZR = NP // NSUB


def _emb_body(ut_ref, emb_ref, out_ref):
    ut = ut_ref[...]
    oh = (ut == lax.broadcasted_iota(jnp.int32, (ut.shape[0], NUM_UNIT), 1))
    out_ref[...] = jnp.dot(oh.astype(jnp.float32), emb_ref[...],
                           preferred_element_type=jnp.float32)


def _embed(ut_col, emb):
    TA = 2560
    return pl.pallas_call(
        _emb_body,
        grid=(NP // TA,),
        in_specs=[
            pl.BlockSpec((TA, 1), lambda i: (i, 0)),
            pl.BlockSpec((NUM_UNIT, H), lambda i: (0, 0)),
        ],
        out_specs=pl.BlockSpec((TA, H), lambda i: (i, 0)),
        out_shape=jax.ShapeDtypeStruct((NP, H), jnp.float32),
    )(ut_col, emb)


def _gather_sc(table, nin2d):
    mesh = plsc.VectorSubcoreMesh(core_axis_name="c", subcore_axis_name="s")
    cpw = (NCHUNKS + NCORE * NSUB - 1) // (NCORE * NSUB)

    @functools.partial(
        pl.kernel, mesh=mesh,
        out_type=jax.ShapeDtypeStruct((E, H), jnp.float32),
        compiler_params=pltpu.CompilerParams(use_tc_tiling_on_sc=False),
        scratch_types=[
            pltpu.VMEM((CH,), jnp.int32),
            pltpu.VMEM((CH, H), jnp.float32),
            pltpu.SemaphoreType.DMA,
        ],
    )
    def k(table_hbm, nin_hbm, out_hbm, idx_v, rows_v, sem):
        wid = lax.axis_index("c") * NSUB + lax.axis_index("s")

        def body(j, carry):
            chunk = wid + j * (NCORE * NSUB)

            @pl.when(chunk < NCHUNKS)
            def _():
                pltpu.sync_copy(nin_hbm.at[chunk], idx_v)
                pltpu.async_copy(table_hbm.at[idx_v], rows_v, sem).wait()
                pltpu.sync_copy(rows_v, out_hbm.at[pl.ds(chunk * CH, CH)])

            return carry

        lax.fori_loop(0, cpw, body, 0)

    return k(table, nin2d)


def _edge_body(eaT_ref, xs_ref, w1T_ref, b1c_ref, w2T_ref, b2T_ref, out_ref):
    eaT = eaT_ref[...]
    xT = xs_ref[...].T
    hmidT = jnp.maximum(
        jnp.dot(w1T_ref[...], eaT, preferred_element_type=jnp.float32)
        + b1c_ref[:, 0:1], 0.0)
    pT = jnp.concatenate(
        [xT[j:j + 1, :] * hmidT for j in range(H)], axis=0)
    msgT = jnp.dot(w2T_ref[...], pT, preferred_element_type=jnp.float32) \
        + jnp.dot(b2T_ref[...], xT, preferred_element_type=jnp.float32)
    out_ref[...] = msgT.T


def _edge_msg(edge_attrT, x_src, w1T, b1c, w2T, b2T):
    TE = 1280
    return pl.pallas_call(
        _edge_body,
        grid=(E // TE,),
        in_specs=[
            pl.BlockSpec((ED, TE), lambda i: (0, i)),
            pl.BlockSpec((TE, H), lambda i: (i, 0)),
            pl.BlockSpec((H, ED), lambda i: (0, 0)),
            pl.BlockSpec((H, 8), lambda i: (0, 0)),
            pl.BlockSpec((H, H * H), lambda i: (0, 0)),
            pl.BlockSpec((H, H), lambda i: (0, 0)),
        ],
        out_specs=pl.BlockSpec((TE, H), lambda i: (i, 0)),
        out_shape=jax.ShapeDtypeStruct((E, H), jnp.float32),
    )(edge_attrT, x_src, w1T, b1c, w2T, b2T)


def _scatter_sc(msg, nout2d, zeros_np):
    mesh = plsc.VectorSubcoreMesh(core_axis_name="c", subcore_axis_name="s")
    half = NCHUNKS // NCORE
    cpw = (half + NSUB - 1) // NSUB

    @functools.partial(
        pl.kernel, mesh=mesh,
        out_type=jax.ShapeDtypeStruct((NCORE, NP, H), jnp.float32),
        compiler_params=pltpu.CompilerParams(use_tc_tiling_on_sc=False),
        scratch_types=[
            pltpu.VMEM((CH,), jnp.int32),
            pltpu.VMEM((CH, H), jnp.float32),
            pltpu.VMEM_SHARED((NP, H), jnp.float32),
        ],
    )
    def k(msg_hbm, nout_hbm, zeros_hbm, parts_hbm, idx_v, rows_v, acc_sh):
        c = lax.axis_index("c")
        s = lax.axis_index("s")
        pltpu.sync_copy(zeros_hbm.at[pl.ds(s * ZR, ZR)],
                        acc_sh.at[pl.ds(s * ZR, ZR)])
        plsc.subcore_barrier()

        def body(j, carry):
            chunk = c * half + s + j * NSUB

            @pl.when(chunk < (c + 1) * half)
            def _():
                pltpu.sync_copy(nout_hbm.at[chunk], idx_v)
                pltpu.sync_copy(msg_hbm.at[pl.ds(chunk * CH, CH)], rows_v)
                pltpu.sync_copy(rows_v, acc_sh.at[idx_v], add=True)

            return carry

        lax.fori_loop(0, cpw, body, 0)
        plsc.subcore_barrier()
        pltpu.sync_copy(acc_sh.at[pl.ds(s * ZR, ZR)],
                        parts_hbm.at[c, pl.ds(s * ZR, ZR)])

    return k(msg, nout2d, zeros_np)


def _gru_body(up_ref, li_ref, giT_ref, ghT_ref, gb_ref, nf_ref):
    x = jnp.maximum(up_ref[0] + up_ref[1], 0.0)
    hx = li_ref[...]
    gi = jnp.dot(x, giT_ref[...], preferred_element_type=jnp.float32) \
        + gb_ref[0:1, :]
    gh = jnp.dot(hx, ghT_ref[...], preferred_element_type=jnp.float32) \
        + gb_ref[1:2, :]
    r = jax.nn.sigmoid(gi[:, :H] + gh[:, :H])
    z = jax.nn.sigmoid(gi[:, H:2 * H] + gh[:, H:2 * H])
    nn_ = jnp.tanh(gi[:, 2 * H:] + r * gh[:, 2 * H:])
    nf_ref[...] = (1.0 - z) * nn_ + z * hx


def _gru(parts, li, giT, ghT, gb):
    TN = 2560
    return pl.pallas_call(
        _gru_body,
        grid=(NP // TN,),
        in_specs=[
            pl.BlockSpec((2, TN, H), lambda i: (0, i, 0)),
            pl.BlockSpec((TN, H), lambda i: (i, 0)),
            pl.BlockSpec((H, 3 * H), lambda i: (0, 0)),
            pl.BlockSpec((H, 3 * H), lambda i: (0, 0)),
            pl.BlockSpec((8, 3 * H), lambda i: (0, 0)),
        ],
        out_specs=pl.BlockSpec((TN, H), lambda i: (i, 0)),
        out_shape=jax.ShapeDtypeStruct((NP, H), jnp.float32),
    )(parts, li, giT, ghT, gb)


def _s2s_body(nf_ref, nfT_ref, n2gr_ref, liT_ref, lhT_ref, lb_ref, qs_ref):
    nf = nf_ref[...]
    nfT = nfT_ref[...]
    n2gr = n2gr_ref[0:1, :]
    ohT = (n2gr == lax.broadcasted_iota(jnp.int32, (B, NP), 0))
    ohf = ohT.astype(jnp.float32)

    h_l = jnp.zeros((B, H), jnp.float32)
    c_l = jnp.zeros((B, H), jnp.float32)
    q_star = jnp.zeros((B, 2 * H), jnp.float32)
    for _ in range(3):
        gates = jnp.dot(q_star, liT_ref[...], preferred_element_type=jnp.float32) \
            + jnp.dot(h_l, lhT_ref[...], preferred_element_type=jnp.float32) \
            + lb_ref[0:1, :]
        ig = jax.nn.sigmoid(gates[:, :H])
        fg = jax.nn.sigmoid(gates[:, H:2 * H])
        gg = jnp.tanh(gates[:, 2 * H:3 * H])
        og = jax.nn.sigmoid(gates[:, 3 * H:])
        c_l = fg * c_l + ig * gg
        h_l = og * jnp.tanh(c_l)
        q = h_l
        s = jnp.dot(q, nfT, preferred_element_type=jnp.float32)
        product = jnp.sum(ohf * s, axis=0, keepdims=True)
        pmax_b = jnp.max(jnp.where(ohT, product, -1e30), axis=1,
                         keepdims=True)
        pmax_n = jnp.sum(jnp.where(ohT, pmax_b, 0.0), axis=0,
                         keepdims=True)
        pexp = jnp.exp(product - pmax_n)
        psum_b = jnp.sum(ohf * pexp, axis=1, keepdims=True)
        psum_n = jnp.sum(jnp.where(ohT, psum_b, 0.0), axis=0,
                         keepdims=True)
        att = pexp / (psum_n + 1e-10)
        out = jnp.dot(ohf * att, nf, preferred_element_type=jnp.float32)
        q_star = jnp.concatenate([q, out], axis=1)
    qs_ref[...] = q_star


def _s2s(nf, nfT, n2gr, liT, lhT, lb):
    return pl.pallas_call(
        _s2s_body,
        out_shape=jax.ShapeDtypeStruct((B, 2 * H), jnp.float32),
        compiler_params=pltpu.CompilerParams(
            vmem_limit_bytes=64 * 1024 * 1024),
    )(nf, nfT, n2gr, liT, lhT, lb)


def kernel(unit_type, edge_index, edge_attr, node2graph, input, emb,
           mlp_w1, mlp_b1, mlp_w2, mlp_b2,
           gru_w_ih, gru_w_hh, gru_b_ih, gru_b_hh,
           lstm_w_ih, lstm_w_hh, lstm_b_ih, lstm_b_hh):
    ut_col = jnp.concatenate(
        [unit_type.astype(jnp.int32), jnp.zeros((NP - N,), jnp.int32)]
    ).reshape(NP, 1)
    n2g = jnp.concatenate(
        [node2graph.astype(jnp.int32), jnp.full((NP - N,), B, jnp.int32)])
    n2gr = jnp.broadcast_to(n2g.reshape(1, NP), (8, NP))

    nin2d = edge_index[0].astype(jnp.int32).reshape(NCHUNKS, CH)
    nout2d = edge_index[1].astype(jnp.int32).reshape(NCHUNKS, CH)

    w2T = mlp_w2.reshape(H, H, H).transpose(1, 2, 0).reshape(H, H * H)
    b2T = mlp_b2.reshape(H, H)
    w1T = mlp_w1.T
    b1c = jnp.broadcast_to(mlp_b1.reshape(H, 1), (H, 8))
    eaT = edge_attr.T

    giT = gru_w_ih.T
    ghT = gru_w_hh.T
    gb = jnp.concatenate(
        [gru_b_ih.reshape(1, 3 * H), gru_b_hh.reshape(1, 3 * H)], axis=0)
    gb = jnp.concatenate([gb, jnp.zeros((6, 3 * H), jnp.float32)], axis=0)
    liT = lstm_w_ih.T
    lhT = lstm_w_hh.T
    lb = jnp.broadcast_to(
        (lstm_b_ih + lstm_b_hh).reshape(1, 4 * H), (8, 4 * H))

    layer_input = _embed(ut_col, emb)
    x_src = _gather_sc(layer_input, nin2d)
    msg = _edge_msg(eaT, x_src, w1T, b1c, w2T, b2T)
    parts = _scatter_sc(msg, nout2d, jnp.zeros((NP, H), jnp.float32))
    nf = _gru(parts, layer_input, giT, ghT, gb)
    q_star = _s2s(nf, nf.T, n2gr, liT, lhT, lb)
    return (q_star, nf[:N])


# speedup vs baseline: 6.2212x; 1.1226x over previous
import functools

import jax
import jax.numpy as jnp
from jax import lax
from jax.experimental import pallas as pl
from jax.experimental.pallas import tpu as pltpu
from jax.experimental.pallas import tpu_sc as plsc

N = 10000
E = 160000
H = 32
ED = 16
NUM_UNIT = 256
B = 128
NP = 10240
CH = 128
NCHUNKS = E // CH
NSUB = 16
NCORE = 2
ZR = NP // NSUB


def _emb_body(ut_ref, emb_ref, out_ref):
    ut = ut_ref[...]
    oh = (ut == lax.broadcasted_iota(jnp.int32, (ut.shape[0], NUM_UNIT), 1))
    out_ref[...] = jnp.dot(oh.astype(jnp.float32), emb_ref[...],
                           preferred_element_type=jnp.float32)


def _embed(ut_col, emb):
    TA = 2560
    return pl.pallas_call(
        _emb_body,
        grid=(NP // TA,),
        in_specs=[
            pl.BlockSpec((TA, 1), lambda i: (i, 0)),
            pl.BlockSpec((NUM_UNIT, H), lambda i: (0, 0)),
        ],
        out_specs=pl.BlockSpec((TA, H), lambda i: (i, 0)),
        out_shape=jax.ShapeDtypeStruct((NP, H), jnp.float32),
    )(ut_col, emb)


def _gather_sc(table, nin2d):
    mesh = plsc.VectorSubcoreMesh(core_axis_name="c", subcore_axis_name="s")
    cpw = (NCHUNKS + NCORE * NSUB - 1) // (NCORE * NSUB)

    nw = NCORE * NSUB

    @functools.partial(
        pl.kernel, mesh=mesh,
        out_type=jax.ShapeDtypeStruct((E, H), jnp.float32),
        compiler_params=pltpu.CompilerParams(use_tc_tiling_on_sc=False),
        scratch_types=[
            pltpu.VMEM((CH,), jnp.int32),
            pltpu.VMEM((CH,), jnp.int32),
            pltpu.VMEM((CH, H), jnp.float32),
            pltpu.VMEM((CH, H), jnp.float32),
            pltpu.SemaphoreType.DMA,
            pltpu.SemaphoreType.DMA,
        ],
    )
    def k(table_hbm, nin_hbm, out_hbm, idx0, idx1, rows0, rows1, sem0, sem1):
        idx = (idx0, idx1)
        rows = (rows0, rows1)
        sems = (sem0, sem1)
        wid = lax.axis_index("c") * NSUB + lax.axis_index("s")

        for b in range(2):
            first = wid + b * nw

            @pl.when(first < NCHUNKS)
            def _(b=b, first=first):
                pltpu.sync_copy(nin_hbm.at[first], idx[b])
                pltpu.async_copy(table_hbm.at[idx[b]], rows[b], sems[b])

        def body(g, carry):
            for b in range(2):
                chunk = wid + (g * 2 + b) * nw

                @pl.when(chunk < NCHUNKS)
                def _(b=b, chunk=chunk):
                    pltpu.make_async_copy(
                        table_hbm.at[idx[b]], rows[b], sems[b]).wait()
                    pltpu.sync_copy(rows[b], out_hbm.at[pl.ds(chunk * CH, CH)])
                    nxt = chunk + 2 * nw

                    @pl.when(nxt < NCHUNKS)
                    def _():
                        pltpu.sync_copy(nin_hbm.at[nxt], idx[b])
                        pltpu.async_copy(table_hbm.at[idx[b]], rows[b], sems[b])

            return carry

        lax.fori_loop(0, (cpw + 1) // 2, body, 0)

    return k(table, nin2d)


def _edge_body(eaT_ref, xs_ref, w1T_ref, b1c_ref, w2T_ref, b2T_ref, out_ref):
    eaT = eaT_ref[...]
    xT = xs_ref[...].T
    hmidT = jnp.maximum(
        jnp.dot(w1T_ref[...], eaT, preferred_element_type=jnp.float32)
        + b1c_ref[:, 0:1], 0.0)
    pT = jnp.concatenate(
        [xT[j:j + 1, :] * hmidT for j in range(H)], axis=0)
    msgT = jnp.dot(w2T_ref[...], pT, preferred_element_type=jnp.float32) \
        + jnp.dot(b2T_ref[...], xT, preferred_element_type=jnp.float32)
    out_ref[...] = msgT.T


def _edge_msg(edge_attrT, x_src, w1T, b1c, w2T, b2T):
    TE = 1280
    return pl.pallas_call(
        _edge_body,
        grid=(E // TE,),
        in_specs=[
            pl.BlockSpec((ED, TE), lambda i: (0, i)),
            pl.BlockSpec((TE, H), lambda i: (i, 0)),
            pl.BlockSpec((H, ED), lambda i: (0, 0)),
            pl.BlockSpec((H, 8), lambda i: (0, 0)),
            pl.BlockSpec((H, H * H), lambda i: (0, 0)),
            pl.BlockSpec((H, H), lambda i: (0, 0)),
        ],
        out_specs=pl.BlockSpec((TE, H), lambda i: (i, 0)),
        out_shape=jax.ShapeDtypeStruct((E, H), jnp.float32),
    )(edge_attrT, x_src, w1T, b1c, w2T, b2T)


def _scatter_sc(msg, nout2d, zeros_np):
    mesh = plsc.VectorSubcoreMesh(core_axis_name="c", subcore_axis_name="s")
    half = NCHUNKS // NCORE
    cpw = (half + NSUB - 1) // NSUB

    @functools.partial(
        pl.kernel, mesh=mesh,
        out_type=jax.ShapeDtypeStruct((NCORE, NP, H), jnp.float32),
        compiler_params=pltpu.CompilerParams(use_tc_tiling_on_sc=False),
        scratch_types=[
            pltpu.VMEM((CH,), jnp.int32),
            pltpu.VMEM((CH,), jnp.int32),
            pltpu.VMEM((CH, H), jnp.float32),
            pltpu.VMEM((CH, H), jnp.float32),
            pltpu.VMEM_SHARED((NP, H), jnp.float32),
            pltpu.SemaphoreType.DMA,
            pltpu.SemaphoreType.DMA,
        ],
    )
    def k(msg_hbm, nout_hbm, zeros_hbm, parts_hbm,
          idx0, idx1, rows0, rows1, acc_sh, sem0, sem1):
        idx = (idx0, idx1)
        rows = (rows0, rows1)
        sems = (sem0, sem1)
        c = lax.axis_index("c")
        s = lax.axis_index("s")
        pltpu.sync_copy(zeros_hbm.at[pl.ds(s * ZR, ZR)],
                        acc_sh.at[pl.ds(s * ZR, ZR)])

        for b in range(2):
            first = c * half + s + b * NSUB

            @pl.when(first < (c + 1) * half)
            def _(b=b, first=first):
                pltpu.sync_copy(nout_hbm.at[first], idx[b])
                pltpu.async_copy(
                    msg_hbm.at[pl.ds(first * CH, CH)], rows[b], sems[b])

        plsc.subcore_barrier()

        def body(g, carry):
            for b in range(2):
                chunk = c * half + s + (g * 2 + b) * NSUB

                @pl.when(chunk < (c + 1) * half)
                def _(b=b, chunk=chunk):
                    pltpu.make_async_copy(
                        msg_hbm.at[pl.ds(chunk * CH, CH)], rows[b],
                        sems[b]).wait()
                    pltpu.sync_copy(rows[b], acc_sh.at[idx[b]], add=True)
                    nxt = chunk + 2 * NSUB

                    @pl.when(nxt < (c + 1) * half)
                    def _():
                        pltpu.sync_copy(nout_hbm.at[nxt], idx[b])
                        pltpu.async_copy(
                            msg_hbm.at[pl.ds(nxt * CH, CH)], rows[b], sems[b])

            return carry

        lax.fori_loop(0, (cpw + 1) // 2, body, 0)
        plsc.subcore_barrier()
        pltpu.sync_copy(acc_sh.at[pl.ds(s * ZR, ZR)],
                        parts_hbm.at[c, pl.ds(s * ZR, ZR)])

    return k(msg, nout2d, zeros_np)


def _gru_body(up_ref, li_ref, giT_ref, ghT_ref, gb_ref, nf_ref):
    x = jnp.maximum(up_ref[0] + up_ref[1], 0.0)
    hx = li_ref[...]
    gi = jnp.dot(x, giT_ref[...], preferred_element_type=jnp.float32) \
        + gb_ref[0:1, :]
    gh = jnp.dot(hx, ghT_ref[...], preferred_element_type=jnp.float32) \
        + gb_ref[1:2, :]
    r = jax.nn.sigmoid(gi[:, :H] + gh[:, :H])
    z = jax.nn.sigmoid(gi[:, H:2 * H] + gh[:, H:2 * H])
    nn_ = jnp.tanh(gi[:, 2 * H:] + r * gh[:, 2 * H:])
    nf_ref[...] = (1.0 - z) * nn_ + z * hx


def _gru(parts, li, giT, ghT, gb):
    TN = 2560
    return pl.pallas_call(
        _gru_body,
        grid=(NP // TN,),
        in_specs=[
            pl.BlockSpec((2, TN, H), lambda i: (0, i, 0)),
            pl.BlockSpec((TN, H), lambda i: (i, 0)),
            pl.BlockSpec((H, 3 * H), lambda i: (0, 0)),
            pl.BlockSpec((H, 3 * H), lambda i: (0, 0)),
            pl.BlockSpec((8, 3 * H), lambda i: (0, 0)),
        ],
        out_specs=pl.BlockSpec((TN, H), lambda i: (i, 0)),
        out_shape=jax.ShapeDtypeStruct((NP, H), jnp.float32),
    )(parts, li, giT, ghT, gb)


def _s2s_body(nf_ref, nfT_ref, n2gr_ref, liT_ref, lhT_ref, lb_ref, qs_ref):
    nf = nf_ref[...]
    nfT = nfT_ref[...]
    n2gr = n2gr_ref[0:1, :]
    ohT = (n2gr == lax.broadcasted_iota(jnp.int32, (B, NP), 0))
    ohf = ohT.astype(jnp.float32)

    h_l = jnp.zeros((B, H), jnp.float32)
    c_l = jnp.zeros((B, H), jnp.float32)
    q_star = jnp.zeros((B, 2 * H), jnp.float32)
    for _ in range(3):
        gates = jnp.dot(q_star, liT_ref[...], preferred_element_type=jnp.float32) \
            + jnp.dot(h_l, lhT_ref[...], preferred_element_type=jnp.float32) \
            + lb_ref[0:1, :]
        ig = jax.nn.sigmoid(gates[:, :H])
        fg = jax.nn.sigmoid(gates[:, H:2 * H])
        gg = jnp.tanh(gates[:, 2 * H:3 * H])
        og = jax.nn.sigmoid(gates[:, 3 * H:])
        c_l = fg * c_l + ig * gg
        h_l = og * jnp.tanh(c_l)
        q = h_l
        s = jnp.dot(q, nfT, preferred_element_type=jnp.float32)
        product = jnp.sum(ohf * s, axis=0, keepdims=True)
        pmax_b = jnp.max(jnp.where(ohT, product, -1e30), axis=1,
                         keepdims=True)
        pmax_n = jnp.sum(jnp.where(ohT, pmax_b, 0.0), axis=0,
                         keepdims=True)
        pexp = jnp.exp(product - pmax_n)
        psum_b = jnp.sum(ohf * pexp, axis=1, keepdims=True)
        psum_n = jnp.sum(jnp.where(ohT, psum_b, 0.0), axis=0,
                         keepdims=True)
        att = pexp / (psum_n + 1e-10)
        out = jnp.dot(ohf * att, nf, preferred_element_type=jnp.float32)
        q_star = jnp.concatenate([q, out], axis=1)
    qs_ref[...] = q_star


def _s2s(nf, nfT, n2gr, liT, lhT, lb):
    return pl.pallas_call(
        _s2s_body,
        out_shape=jax.ShapeDtypeStruct((B, 2 * H), jnp.float32),
        compiler_params=pltpu.CompilerParams(
            vmem_limit_bytes=64 * 1024 * 1024),
    )(nf, nfT, n2gr, liT, lhT, lb)


def kernel(unit_type, edge_index, edge_attr, node2graph, input, emb,
           mlp_w1, mlp_b1, mlp_w2, mlp_b2,
           gru_w_ih, gru_w_hh, gru_b_ih, gru_b_hh,
           lstm_w_ih, lstm_w_hh, lstm_b_ih, lstm_b_hh):
    ut_col = jnp.concatenate(
        [unit_type.astype(jnp.int32), jnp.zeros((NP - N,), jnp.int32)]
    ).reshape(NP, 1)
    n2g = jnp.concatenate(
        [node2graph.astype(jnp.int32), jnp.full((NP - N,), B, jnp.int32)])
    n2gr = jnp.broadcast_to(n2g.reshape(1, NP), (8, NP))

    nin2d = edge_index[0].astype(jnp.int32).reshape(NCHUNKS, CH)
    nout2d = edge_index[1].astype(jnp.int32).reshape(NCHUNKS, CH)

    w2T = mlp_w2.reshape(H, H, H).transpose(1, 2, 0).reshape(H, H * H)
    b2T = mlp_b2.reshape(H, H)
    w1T = mlp_w1.T
    b1c = jnp.broadcast_to(mlp_b1.reshape(H, 1), (H, 8))
    eaT = edge_attr.T

    giT = gru_w_ih.T
    ghT = gru_w_hh.T
    gb = jnp.concatenate(
        [gru_b_ih.reshape(1, 3 * H), gru_b_hh.reshape(1, 3 * H)], axis=0)
    gb = jnp.concatenate([gb, jnp.zeros((6, 3 * H), jnp.float32)], axis=0)
    liT = lstm_w_ih.T
    lhT = lstm_w_hh.T
    lb = jnp.broadcast_to(
        (lstm_b_ih + lstm_b_hh).reshape(1, 4 * H), (8, 4 * H))

    layer_input = _embed(ut_col, emb)
    x_src = _gather_sc(layer_input, nin2d)
    msg = _edge_msg(eaT, x_src, w1T, b1c, w2T, b2T)
    parts = _scatter_sc(msg, nout2d, jnp.zeros((NP, H), jnp.float32))
    nf = _gru(parts, layer_input, giT, ghT, gb)
    q_star = _s2s(nf, nf.T, n2gr, liT, lhT, lb)
    return (q_star, nf[:N])


# speedup vs baseline: 6.3935x; 1.0277x over previous
import functools

import jax
import jax.numpy as jnp
from jax import lax
from jax.experimental import pallas as pl
from jax.experimental.pallas import tpu as pltpu
from jax.experimental.pallas import tpu_sc as plsc

N = 10000
E = 160000
H = 32
ED = 16
NUM_UNIT = 256
B = 128
NP = 10240
CH = 128
NCHUNKS = E // CH
NSUB = 16
NCORE = 2
ZR = NP // NSUB


def _emb_body(ut_ref, emb_ref, out_ref):
    ut = ut_ref[...]
    oh = (ut == lax.broadcasted_iota(jnp.int32, (ut.shape[0], NUM_UNIT), 1))
    out_ref[...] = jnp.dot(oh.astype(jnp.float32), emb_ref[...],
                           preferred_element_type=jnp.float32)


def _embed(ut_col, emb):
    TA = 2560
    return pl.pallas_call(
        _emb_body,
        grid=(NP // TA,),
        in_specs=[
            pl.BlockSpec((TA, 1), lambda i: (i, 0)),
            pl.BlockSpec((NUM_UNIT, H), lambda i: (0, 0)),
        ],
        out_specs=pl.BlockSpec((TA, H), lambda i: (i, 0)),
        out_shape=jax.ShapeDtypeStruct((NP, H), jnp.float32),
    )(ut_col, emb)


def _gather_sc(table, nin2d):
    mesh = plsc.VectorSubcoreMesh(core_axis_name="c", subcore_axis_name="s")
    nchunks = nin2d.shape[0]
    nw = NCORE * NSUB
    cpw = (nchunks + nw - 1) // nw

    @functools.partial(
        pl.kernel, mesh=mesh,
        out_type=jax.ShapeDtypeStruct((nchunks * CH, H), jnp.float32),
        compiler_params=pltpu.CompilerParams(use_tc_tiling_on_sc=False),
        scratch_types=[
            pltpu.VMEM((CH,), jnp.int32),
            pltpu.VMEM((CH,), jnp.int32),
            pltpu.VMEM((CH, H), jnp.float32),
            pltpu.VMEM((CH, H), jnp.float32),
            pltpu.SemaphoreType.DMA,
            pltpu.SemaphoreType.DMA,
        ],
    )
    def k(table_hbm, nin_hbm, out_hbm, idx0, idx1, rows0, rows1, sem0, sem1):
        idx = (idx0, idx1)
        rows = (rows0, rows1)
        sems = (sem0, sem1)
        wid = lax.axis_index("c") * NSUB + lax.axis_index("s")

        for b in range(2):
            first = wid + b * nw

            @pl.when(first < nchunks)
            def _(b=b, first=first):
                pltpu.sync_copy(nin_hbm.at[first], idx[b])
                pltpu.async_copy(table_hbm.at[idx[b]], rows[b], sems[b])

        def body(g, carry):
            for b in range(2):
                chunk = wid + (g * 2 + b) * nw

                @pl.when(chunk < nchunks)
                def _(b=b, chunk=chunk):
                    pltpu.make_async_copy(
                        table_hbm.at[idx[b]], rows[b], sems[b]).wait()
                    pltpu.sync_copy(rows[b], out_hbm.at[pl.ds(chunk * CH, CH)])
                    nxt = chunk + 2 * nw

                    @pl.when(nxt < nchunks)
                    def _():
                        pltpu.sync_copy(nin_hbm.at[nxt], idx[b])
                        pltpu.async_copy(table_hbm.at[idx[b]], rows[b], sems[b])

            return carry

        lax.fori_loop(0, (cpw + 1) // 2, body, 0)

    return k(table, nin2d)


def _edge_body(eaT_ref, xs_ref, w1T_ref, b1c_ref, w2T_ref, b2T_ref, out_ref):
    eaT = eaT_ref[...]
    xT = xs_ref[...].T
    hmidT = jnp.maximum(
        jnp.dot(w1T_ref[...], eaT, preferred_element_type=jnp.float32)
        + b1c_ref[:, 0:1], 0.0)
    pT = jnp.concatenate(
        [xT[j:j + 1, :] * hmidT for j in range(H)], axis=0)
    msgT = jnp.dot(w2T_ref[...], pT, preferred_element_type=jnp.float32) \
        + jnp.dot(b2T_ref[...], xT, preferred_element_type=jnp.float32)
    out_ref[...] = msgT.T


def _edge_msg(edge_attrT, x_src, w1T, b1c, w2T, b2T):
    TE = 1280
    e = x_src.shape[0]
    return pl.pallas_call(
        _edge_body,
        grid=(e // TE,),
        in_specs=[
            pl.BlockSpec((ED, TE), lambda i: (0, i)),
            pl.BlockSpec((TE, H), lambda i: (i, 0)),
            pl.BlockSpec((H, ED), lambda i: (0, 0)),
            pl.BlockSpec((H, 8), lambda i: (0, 0)),
            pl.BlockSpec((H, H * H), lambda i: (0, 0)),
            pl.BlockSpec((H, H), lambda i: (0, 0)),
        ],
        out_specs=pl.BlockSpec((TE, H), lambda i: (i, 0)),
        out_shape=jax.ShapeDtypeStruct((e, H), jnp.float32),
    )(edge_attrT, x_src, w1T, b1c, w2T, b2T)


def _scatter_sc(msg, nout2d, zeros_np):
    mesh = plsc.VectorSubcoreMesh(core_axis_name="c", subcore_axis_name="s")
    half = nout2d.shape[0] // NCORE
    cpw = (half + NSUB - 1) // NSUB

    @functools.partial(
        pl.kernel, mesh=mesh,
        out_type=jax.ShapeDtypeStruct((NCORE, NP, H), jnp.float32),
        compiler_params=pltpu.CompilerParams(use_tc_tiling_on_sc=False),
        scratch_types=[
            pltpu.VMEM((CH,), jnp.int32),
            pltpu.VMEM((CH,), jnp.int32),
            pltpu.VMEM((CH, H), jnp.float32),
            pltpu.VMEM((CH, H), jnp.float32),
            pltpu.VMEM_SHARED((NP, H), jnp.float32),
            pltpu.SemaphoreType.DMA,
            pltpu.SemaphoreType.DMA,
        ],
    )
    def k(msg_hbm, nout_hbm, zeros_hbm, parts_hbm,
          idx0, idx1, rows0, rows1, acc_sh, sem0, sem1):
        idx = (idx0, idx1)
        rows = (rows0, rows1)
        sems = (sem0, sem1)
        c = lax.axis_index("c")
        s = lax.axis_index("s")
        pltpu.sync_copy(zeros_hbm.at[pl.ds(s * ZR, ZR)],
                        acc_sh.at[pl.ds(s * ZR, ZR)])

        for b in range(2):
            first = c * half + s + b * NSUB

            @pl.when(first < (c + 1) * half)
            def _(b=b, first=first):
                pltpu.sync_copy(nout_hbm.at[first], idx[b])
                pltpu.async_copy(
                    msg_hbm.at[pl.ds(first * CH, CH)], rows[b], sems[b])

        plsc.subcore_barrier()

        def body(g, carry):
            for b in range(2):
                chunk = c * half + s + (g * 2 + b) * NSUB

                @pl.when(chunk < (c + 1) * half)
                def _(b=b, chunk=chunk):
                    pltpu.make_async_copy(
                        msg_hbm.at[pl.ds(chunk * CH, CH)], rows[b],
                        sems[b]).wait()
                    pltpu.sync_copy(rows[b], acc_sh.at[idx[b]], add=True)
                    nxt = chunk + 2 * NSUB

                    @pl.when(nxt < (c + 1) * half)
                    def _():
                        pltpu.sync_copy(nout_hbm.at[nxt], idx[b])
                        pltpu.async_copy(
                            msg_hbm.at[pl.ds(nxt * CH, CH)], rows[b], sems[b])

            return carry

        lax.fori_loop(0, (cpw + 1) // 2, body, 0)
        plsc.subcore_barrier()
        pltpu.sync_copy(acc_sh.at[pl.ds(s * ZR, ZR)],
                        parts_hbm.at[c, pl.ds(s * ZR, ZR)])

    return k(msg, nout2d, zeros_np)


def _gru_body(up_ref, up2_ref, li_ref, giT_ref, ghT_ref, gb_ref, nf_ref):
    x = jnp.maximum(up_ref[0] + up_ref[1] + up2_ref[0] + up2_ref[1], 0.0)
    hx = li_ref[...]
    gi = jnp.dot(x, giT_ref[...], preferred_element_type=jnp.float32) \
        + gb_ref[0:1, :]
    gh = jnp.dot(hx, ghT_ref[...], preferred_element_type=jnp.float32) \
        + gb_ref[1:2, :]
    r = jax.nn.sigmoid(gi[:, :H] + gh[:, :H])
    z = jax.nn.sigmoid(gi[:, H:2 * H] + gh[:, H:2 * H])
    nn_ = jnp.tanh(gi[:, 2 * H:] + r * gh[:, 2 * H:])
    nf_ref[...] = (1.0 - z) * nn_ + z * hx


def _gru(parts, parts2, li, giT, ghT, gb):
    TN = 2560
    return pl.pallas_call(
        _gru_body,
        grid=(NP // TN,),
        in_specs=[
            pl.BlockSpec((2, TN, H), lambda i: (0, i, 0)),
            pl.BlockSpec((2, TN, H), lambda i: (0, i, 0)),
            pl.BlockSpec((TN, H), lambda i: (i, 0)),
            pl.BlockSpec((H, 3 * H), lambda i: (0, 0)),
            pl.BlockSpec((H, 3 * H), lambda i: (0, 0)),
            pl.BlockSpec((8, 3 * H), lambda i: (0, 0)),
        ],
        out_specs=pl.BlockSpec((TN, H), lambda i: (i, 0)),
        out_shape=jax.ShapeDtypeStruct((NP, H), jnp.float32),
    )(parts, parts2, li, giT, ghT, gb)


def _s2s_body(nf_ref, nfT_ref, n2gr_ref, liT_ref, lhT_ref, lb_ref, qs_ref):
    nf = nf_ref[...]
    nfT = nfT_ref[...]
    n2gr = n2gr_ref[0:1, :]
    ohT = (n2gr == lax.broadcasted_iota(jnp.int32, (B, NP), 0))
    ohf = ohT.astype(jnp.float32)

    h_l = jnp.zeros((B, H), jnp.float32)
    c_l = jnp.zeros((B, H), jnp.float32)
    q_star = jnp.zeros((B, 2 * H), jnp.float32)
    for _ in range(3):
        gates = jnp.dot(q_star, liT_ref[...], preferred_element_type=jnp.float32) \
            + jnp.dot(h_l, lhT_ref[...], preferred_element_type=jnp.float32) \
            + lb_ref[0:1, :]
        ig = jax.nn.sigmoid(gates[:, :H])
        fg = jax.nn.sigmoid(gates[:, H:2 * H])
        gg = jnp.tanh(gates[:, 2 * H:3 * H])
        og = jax.nn.sigmoid(gates[:, 3 * H:])
        c_l = fg * c_l + ig * gg
        h_l = og * jnp.tanh(c_l)
        q = h_l
        s = jnp.dot(q, nfT, preferred_element_type=jnp.float32)
        product = jnp.sum(ohf * s, axis=0, keepdims=True)
        pmax_b = jnp.max(jnp.where(ohT, product, -1e30), axis=1,
                         keepdims=True)
        pmax_n = jnp.sum(jnp.where(ohT, pmax_b, 0.0), axis=0,
                         keepdims=True)
        pexp = jnp.exp(product - pmax_n)
        psum_b = jnp.sum(ohf * pexp, axis=1, keepdims=True)
        psum_n = jnp.sum(jnp.where(ohT, psum_b, 0.0), axis=0,
                         keepdims=True)
        att = pexp / (psum_n + 1e-10)
        out = jnp.dot(ohf * att, nf, preferred_element_type=jnp.float32)
        q_star = jnp.concatenate([q, out], axis=1)
    qs_ref[...] = q_star


def _s2s(nf, nfT, n2gr, liT, lhT, lb):
    return pl.pallas_call(
        _s2s_body,
        out_shape=jax.ShapeDtypeStruct((B, 2 * H), jnp.float32),
        compiler_params=pltpu.CompilerParams(
            vmem_limit_bytes=64 * 1024 * 1024),
    )(nf, nfT, n2gr, liT, lhT, lb)


def kernel(unit_type, edge_index, edge_attr, node2graph, input, emb,
           mlp_w1, mlp_b1, mlp_w2, mlp_b2,
           gru_w_ih, gru_w_hh, gru_b_ih, gru_b_hh,
           lstm_w_ih, lstm_w_hh, lstm_b_ih, lstm_b_hh):
    ut_col = jnp.concatenate(
        [unit_type.astype(jnp.int32), jnp.zeros((NP - N,), jnp.int32)]
    ).reshape(NP, 1)
    n2g = jnp.concatenate(
        [node2graph.astype(jnp.int32), jnp.full((NP - N,), B, jnp.int32)])
    n2gr = jnp.broadcast_to(n2g.reshape(1, NP), (8, NP))

    nin2d = edge_index[0].astype(jnp.int32).reshape(NCHUNKS, CH)
    nout2d = edge_index[1].astype(jnp.int32).reshape(NCHUNKS, CH)

    w2T = mlp_w2.reshape(H, H, H).transpose(1, 2, 0).reshape(H, H * H)
    b2T = mlp_b2.reshape(H, H)
    w1T = mlp_w1.T
    b1c = jnp.broadcast_to(mlp_b1.reshape(H, 1), (H, 8))
    eaT = edge_attr.T

    giT = gru_w_ih.T
    ghT = gru_w_hh.T
    gb = jnp.concatenate(
        [gru_b_ih.reshape(1, 3 * H), gru_b_hh.reshape(1, 3 * H)], axis=0)
    gb = jnp.concatenate([gb, jnp.zeros((6, 3 * H), jnp.float32)], axis=0)
    liT = lstm_w_ih.T
    lhT = lstm_w_hh.T
    lb = jnp.broadcast_to(
        (lstm_b_ih + lstm_b_hh).reshape(1, 4 * H), (8, 4 * H))

    layer_input = _embed(ut_col, emb)
    E0 = 79360
    C0 = E0 // CH
    zeros_np = jnp.zeros((NP, H), jnp.float32)

    x0 = _gather_sc(layer_input, nin2d[:C0])
    x1 = _gather_sc(layer_input, nin2d[C0:])
    msg0 = _edge_msg(eaT[:, :E0], x0, w1T, b1c, w2T, b2T)
    msg1 = _edge_msg(eaT[:, E0:], x1, w1T, b1c, w2T, b2T)
    parts0 = _scatter_sc(msg0, nout2d[:C0], zeros_np)
    parts1 = _scatter_sc(msg1, nout2d[C0:], zeros_np)
    nf = _gru(parts0, parts1, layer_input, giT, ghT, gb)
    q_star = _s2s(nf, nf.T, n2gr, liT, lhT, lb)
    return (q_star, nf[:N])


# speedup vs baseline: 7.7966x; 1.2195x over previous
import functools

import jax
import jax.numpy as jnp
from jax import lax
from jax.experimental import pallas as pl
from jax.experimental.pallas import tpu as pltpu
from jax.experimental.pallas import tpu_sc as plsc

N = 10000
E = 160000
H = 32
ED = 16
NUM_UNIT = 256
B = 128
NP = 10240
CH = 128
NCHUNKS = E // CH
NSUB = 16
NCORE = 2
ZR = NP // NSUB


DW = 128


def _emb_body(ut_ref, emb_ref, out_ref):
    ut = ut_ref[...]
    oh = (ut == lax.broadcasted_iota(jnp.int32, (ut.shape[0], NUM_UNIT), 1))
    out_ref[...] = jnp.dot(oh.astype(jnp.float32), emb_ref[...],
                           preferred_element_type=jnp.float32)


def _embed(ut_col, emb128):
    TA = 2560
    return pl.pallas_call(
        _emb_body,
        grid=(NP // TA,),
        in_specs=[
            pl.BlockSpec((TA, 1), lambda i: (i, 0)),
            pl.BlockSpec((NUM_UNIT, DW), lambda i: (0, 0)),
        ],
        out_specs=pl.BlockSpec((TA, DW), lambda i: (i, 0)),
        out_shape=jax.ShapeDtypeStruct((NP, DW), jnp.float32),
    )(ut_col, emb128)


def _gather_sc(table, nin2d):
    mesh = plsc.VectorSubcoreMesh(core_axis_name="c", subcore_axis_name="s")
    nchunks = nin2d.shape[0]
    nw = NCORE * NSUB
    cpw = (nchunks + nw - 1) // nw

    @functools.partial(
        pl.kernel, mesh=mesh,
        out_type=jax.ShapeDtypeStruct((nchunks * CH, DW), jnp.float32),
        compiler_params=pltpu.CompilerParams(use_tc_tiling_on_sc=False),
        scratch_types=[
            pltpu.VMEM((CH,), jnp.int32),
            pltpu.VMEM((CH,), jnp.int32),
            pltpu.VMEM((CH, DW), jnp.float32),
            pltpu.VMEM((CH, DW), jnp.float32),
            pltpu.SemaphoreType.DMA,
            pltpu.SemaphoreType.DMA,
        ],
    )
    def k(table_hbm, nin_hbm, out_hbm, idx0, idx1, rows0, rows1, sem0, sem1):
        idx = (idx0, idx1)
        rows = (rows0, rows1)
        sems = (sem0, sem1)
        wid = lax.axis_index("c") * NSUB + lax.axis_index("s")

        for b in range(2):
            first = wid + b * nw

            @pl.when(first < nchunks)
            def _(b=b, first=first):
                pltpu.sync_copy(nin_hbm.at[first], idx[b])
                pltpu.async_copy(table_hbm.at[idx[b]], rows[b], sems[b])

        def body(g, carry):
            for b in range(2):
                chunk = wid + (g * 2 + b) * nw

                @pl.when(chunk < nchunks)
                def _(b=b, chunk=chunk):
                    pltpu.make_async_copy(
                        table_hbm.at[idx[b]], rows[b], sems[b]).wait()
                    pltpu.sync_copy(rows[b], out_hbm.at[pl.ds(chunk * CH, CH)])
                    nxt = chunk + 2 * nw

                    @pl.when(nxt < nchunks)
                    def _():
                        pltpu.sync_copy(nin_hbm.at[nxt], idx[b])
                        pltpu.async_copy(table_hbm.at[idx[b]], rows[b], sems[b])

            return carry

        lax.fori_loop(0, (cpw + 1) // 2, body, 0)

    return k(table, nin2d)


def _edge_body(eaT_ref, xs_ref, w1T_ref, b1c_ref, w2T_ref, b2T_ref, out_ref):
    eaT = eaT_ref[...]
    xT = xs_ref[:, :H].T
    hmidT = jnp.maximum(
        jnp.dot(w1T_ref[...], eaT, preferred_element_type=jnp.float32)
        + b1c_ref[:, 0:1], 0.0)
    pT = jnp.concatenate(
        [xT[j:j + 1, :] * hmidT for j in range(H)], axis=0)
    msgT = jnp.dot(w2T_ref[...], pT, preferred_element_type=jnp.float32) \
        + jnp.dot(b2T_ref[...], xT, preferred_element_type=jnp.float32)
    te = msgT.shape[1]
    out_ref[...] = jnp.concatenate(
        [msgT.T, jnp.zeros((te, DW - H), jnp.float32)], axis=1)


def _edge_msg(edge_attrT, x_src, w1T, b1c, w2T, b2T):
    TE = 1280
    e = x_src.shape[0]
    return pl.pallas_call(
        _edge_body,
        grid=(e // TE,),
        in_specs=[
            pl.BlockSpec((ED, TE), lambda i: (0, i)),
            pl.BlockSpec((TE, DW), lambda i: (i, 0)),
            pl.BlockSpec((H, ED), lambda i: (0, 0)),
            pl.BlockSpec((H, 8), lambda i: (0, 0)),
            pl.BlockSpec((H, H * H), lambda i: (0, 0)),
            pl.BlockSpec((H, H), lambda i: (0, 0)),
        ],
        out_specs=pl.BlockSpec((TE, DW), lambda i: (i, 0)),
        out_shape=jax.ShapeDtypeStruct((e, DW), jnp.float32),
    )(edge_attrT, x_src, w1T, b1c, w2T, b2T)


def _scatter_sc(msg, nout2d, zeros_np):
    mesh = plsc.VectorSubcoreMesh(core_axis_name="c", subcore_axis_name="s")
    half = nout2d.shape[0] // NCORE
    cpw = (half + NSUB - 1) // NSUB

    @functools.partial(
        pl.kernel, mesh=mesh,
        out_type=jax.ShapeDtypeStruct((NCORE, NP, DW), jnp.float32),
        compiler_params=pltpu.CompilerParams(use_tc_tiling_on_sc=False),
        scratch_types=[
            pltpu.VMEM((CH,), jnp.int32),
            pltpu.VMEM((CH,), jnp.int32),
            pltpu.VMEM((CH, DW), jnp.float32),
            pltpu.VMEM((CH, DW), jnp.float32),
            pltpu.VMEM_SHARED((NP, DW), jnp.float32),
            pltpu.SemaphoreType.DMA,
            pltpu.SemaphoreType.DMA,
        ],
    )
    def k(msg_hbm, nout_hbm, zeros_hbm, parts_hbm,
          idx0, idx1, rows0, rows1, acc_sh, sem0, sem1):
        idx = (idx0, idx1)
        rows = (rows0, rows1)
        sems = (sem0, sem1)
        c = lax.axis_index("c")
        s = lax.axis_index("s")
        pltpu.sync_copy(zeros_hbm.at[pl.ds(s * ZR, ZR)],
                        acc_sh.at[pl.ds(s * ZR, ZR)])

        for b in range(2):
            first = c * half + s + b * NSUB

            @pl.when(first < (c + 1) * half)
            def _(b=b, first=first):
                pltpu.sync_copy(nout_hbm.at[first], idx[b])
                pltpu.async_copy(
                    msg_hbm.at[pl.ds(first * CH, CH)], rows[b], sems[b])

        plsc.subcore_barrier()

        def body(g, carry):
            for b in range(2):
                chunk = c * half + s + (g * 2 + b) * NSUB

                @pl.when(chunk < (c + 1) * half)
                def _(b=b, chunk=chunk):
                    pltpu.make_async_copy(
                        msg_hbm.at[pl.ds(chunk * CH, CH)], rows[b],
                        sems[b]).wait()
                    pltpu.sync_copy(rows[b], acc_sh.at[idx[b]], add=True)
                    nxt = chunk + 2 * NSUB

                    @pl.when(nxt < (c + 1) * half)
                    def _():
                        pltpu.sync_copy(nout_hbm.at[nxt], idx[b])
                        pltpu.async_copy(
                            msg_hbm.at[pl.ds(nxt * CH, CH)], rows[b], sems[b])

            return carry

        lax.fori_loop(0, (cpw + 1) // 2, body, 0)
        plsc.subcore_barrier()
        pltpu.sync_copy(acc_sh.at[pl.ds(s * ZR, ZR)],
                        parts_hbm.at[c, pl.ds(s * ZR, ZR)])

    return k(msg, nout2d, zeros_np)


def _gru_body(up_ref, up2_ref, li_ref, giT_ref, ghT_ref, gb_ref, nf_ref):
    x = jnp.maximum(up_ref[0, :, :H] + up_ref[1, :, :H]
                    + up2_ref[0, :, :H] + up2_ref[1, :, :H], 0.0)
    hx = li_ref[:, :H]
    gi = jnp.dot(x, giT_ref[...], preferred_element_type=jnp.float32) \
        + gb_ref[0:1, :]
    gh = jnp.dot(hx, ghT_ref[...], preferred_element_type=jnp.float32) \
        + gb_ref[1:2, :]
    r = jax.nn.sigmoid(gi[:, :H] + gh[:, :H])
    z = jax.nn.sigmoid(gi[:, H:2 * H] + gh[:, H:2 * H])
    nn_ = jnp.tanh(gi[:, 2 * H:] + r * gh[:, 2 * H:])
    nf_ref[...] = (1.0 - z) * nn_ + z * hx


def _gru(parts, parts2, li, giT, ghT, gb):
    TN = 2560
    return pl.pallas_call(
        _gru_body,
        grid=(NP // TN,),
        in_specs=[
            pl.BlockSpec((2, TN, DW), lambda i: (0, i, 0)),
            pl.BlockSpec((2, TN, DW), lambda i: (0, i, 0)),
            pl.BlockSpec((TN, DW), lambda i: (i, 0)),
            pl.BlockSpec((H, 3 * H), lambda i: (0, 0)),
            pl.BlockSpec((H, 3 * H), lambda i: (0, 0)),
            pl.BlockSpec((8, 3 * H), lambda i: (0, 0)),
        ],
        out_specs=pl.BlockSpec((TN, H), lambda i: (i, 0)),
        out_shape=jax.ShapeDtypeStruct((NP, H), jnp.float32),
    )(parts, parts2, li, giT, ghT, gb)


def _s2s_body(nf_ref, nfT_ref, n2gr_ref, liT_ref, lhT_ref, lb_ref, qs_ref):
    nf = nf_ref[...]
    nfT = nfT_ref[...]
    n2gr = n2gr_ref[0:1, :]
    ohT = (n2gr == lax.broadcasted_iota(jnp.int32, (B, NP), 0))
    ohf = ohT.astype(jnp.float32)

    h_l = jnp.zeros((B, H), jnp.float32)
    c_l = jnp.zeros((B, H), jnp.float32)
    q_star = jnp.zeros((B, 2 * H), jnp.float32)
    for _ in range(3):
        gates = jnp.dot(q_star, liT_ref[...], preferred_element_type=jnp.float32) \
            + jnp.dot(h_l, lhT_ref[...], preferred_element_type=jnp.float32) \
            + lb_ref[0:1, :]
        ig = jax.nn.sigmoid(gates[:, :H])
        fg = jax.nn.sigmoid(gates[:, H:2 * H])
        gg = jnp.tanh(gates[:, 2 * H:3 * H])
        og = jax.nn.sigmoid(gates[:, 3 * H:])
        c_l = fg * c_l + ig * gg
        h_l = og * jnp.tanh(c_l)
        q = h_l
        s = jnp.dot(q, nfT, preferred_element_type=jnp.float32)
        product = jnp.sum(ohf * s, axis=0, keepdims=True)
        pmax_b = jnp.max(jnp.where(ohT, product, -1e30), axis=1,
                         keepdims=True)
        pmax_n = jnp.sum(jnp.where(ohT, pmax_b, 0.0), axis=0,
                         keepdims=True)
        pexp = jnp.exp(product - pmax_n)
        psum_b = jnp.sum(ohf * pexp, axis=1, keepdims=True)
        psum_n = jnp.sum(jnp.where(ohT, psum_b, 0.0), axis=0,
                         keepdims=True)
        att = pexp / (psum_n + 1e-10)
        out = jnp.dot(ohf * att, nf, preferred_element_type=jnp.float32)
        q_star = jnp.concatenate([q, out], axis=1)
    qs_ref[...] = q_star


def _s2s(nf, nfT, n2gr, liT, lhT, lb):
    return pl.pallas_call(
        _s2s_body,
        out_shape=jax.ShapeDtypeStruct((B, 2 * H), jnp.float32),
        compiler_params=pltpu.CompilerParams(
            vmem_limit_bytes=64 * 1024 * 1024),
    )(nf, nfT, n2gr, liT, lhT, lb)


def kernel(unit_type, edge_index, edge_attr, node2graph, input, emb,
           mlp_w1, mlp_b1, mlp_w2, mlp_b2,
           gru_w_ih, gru_w_hh, gru_b_ih, gru_b_hh,
           lstm_w_ih, lstm_w_hh, lstm_b_ih, lstm_b_hh):
    ut_col = jnp.concatenate(
        [unit_type.astype(jnp.int32), jnp.zeros((NP - N,), jnp.int32)]
    ).reshape(NP, 1)
    n2g = jnp.concatenate(
        [node2graph.astype(jnp.int32), jnp.full((NP - N,), B, jnp.int32)])
    n2gr = jnp.broadcast_to(n2g.reshape(1, NP), (8, NP))

    nin2d = edge_index[0].astype(jnp.int32).reshape(NCHUNKS, CH)
    nout2d = edge_index[1].astype(jnp.int32).reshape(NCHUNKS, CH)

    w2T = mlp_w2.reshape(H, H, H).transpose(1, 2, 0).reshape(H, H * H)
    b2T = mlp_b2.reshape(H, H)
    w1T = mlp_w1.T
    b1c = jnp.broadcast_to(mlp_b1.reshape(H, 1), (H, 8))
    eaT = edge_attr.T

    giT = gru_w_ih.T
    ghT = gru_w_hh.T
    gb = jnp.concatenate(
        [gru_b_ih.reshape(1, 3 * H), gru_b_hh.reshape(1, 3 * H)], axis=0)
    gb = jnp.concatenate([gb, jnp.zeros((6, 3 * H), jnp.float32)], axis=0)
    liT = lstm_w_ih.T
    lhT = lstm_w_hh.T
    lb = jnp.broadcast_to(
        (lstm_b_ih + lstm_b_hh).reshape(1, 4 * H), (8, 4 * H))

    emb128 = jnp.concatenate(
        [emb, jnp.zeros((NUM_UNIT, DW - H), jnp.float32)], axis=1)
    layer_input = _embed(ut_col, emb128)
    E0 = 79360
    C0 = E0 // CH
    zeros_np = jnp.zeros((NP, DW), jnp.float32)

    x0 = _gather_sc(layer_input, nin2d[:C0])
    x1 = _gather_sc(layer_input, nin2d[C0:])
    msg0 = _edge_msg(eaT[:, :E0], x0, w1T, b1c, w2T, b2T)
    msg1 = _edge_msg(eaT[:, E0:], x1, w1T, b1c, w2T, b2T)
    parts0 = _scatter_sc(msg0, nout2d[:C0], zeros_np)
    parts1 = _scatter_sc(msg1, nout2d[C0:], zeros_np)
    nf = _gru(parts0, parts1, layer_input, giT, ghT, gb)
    q_star = _s2s(nf, nf.T, n2gr, liT, lhT, lb)
    return (q_star, nf[:N])


# speedup vs baseline: 8.2027x; 1.0521x over previous
import functools

import jax
import jax.numpy as jnp
from jax import lax
from jax.experimental import pallas as pl
from jax.experimental.pallas import tpu as pltpu
from jax.experimental.pallas import tpu_sc as plsc

N = 10000
E = 160000
H = 32
ED = 16
NUM_UNIT = 256
B = 128
NP = 10240
CH = 128
NCHUNKS = E // CH
NSUB = 16
NCORE = 2
ZR = NP // NSUB


DW = 128


def _emb_body(ut_ref, emb_ref, out_ref):
    ut = ut_ref[...]
    oh = (ut == lax.broadcasted_iota(jnp.int32, (ut.shape[0], NUM_UNIT), 1))
    out_ref[...] = jnp.dot(oh.astype(jnp.float32), emb_ref[...],
                           preferred_element_type=jnp.float32)


def _embed(ut_col, emb128):
    TA = 2560
    return pl.pallas_call(
        _emb_body,
        grid=(NP // TA,),
        in_specs=[
            pl.BlockSpec((TA, 1), lambda i: (i, 0)),
            pl.BlockSpec((NUM_UNIT, DW), lambda i: (0, 0)),
        ],
        out_specs=pl.BlockSpec((TA, DW), lambda i: (i, 0)),
        out_shape=jax.ShapeDtypeStruct((NP, DW), jnp.float32),
    )(ut_col, emb128)


def _gather_sc(table, nin2d):
    mesh = plsc.VectorSubcoreMesh(core_axis_name="c", subcore_axis_name="s")
    nchunks = nin2d.shape[0]
    nw = NCORE * NSUB
    cpw = (nchunks + nw - 1) // nw

    @functools.partial(
        pl.kernel, mesh=mesh,
        out_type=jax.ShapeDtypeStruct((nchunks * CH, DW), jnp.float32),
        compiler_params=pltpu.CompilerParams(use_tc_tiling_on_sc=False),
        scratch_types=[
            pltpu.VMEM((CH,), jnp.int32),
            pltpu.VMEM((CH,), jnp.int32),
            pltpu.VMEM((CH, DW), jnp.float32),
            pltpu.VMEM((CH, DW), jnp.float32),
            pltpu.SemaphoreType.DMA,
            pltpu.SemaphoreType.DMA,
        ],
    )
    def k(table_hbm, nin_hbm, out_hbm, idx0, idx1, rows0, rows1, sem0, sem1):
        idx = (idx0, idx1)
        rows = (rows0, rows1)
        sems = (sem0, sem1)
        wid = lax.axis_index("c") * NSUB + lax.axis_index("s")

        for b in range(2):
            first = wid + b * nw

            @pl.when(first < nchunks)
            def _(b=b, first=first):
                pltpu.sync_copy(nin_hbm.at[first], idx[b])
                pltpu.async_copy(table_hbm.at[idx[b]], rows[b], sems[b])

        def body(g, carry):
            for b in range(2):
                chunk = wid + (g * 2 + b) * nw

                @pl.when(chunk < nchunks)
                def _(b=b, chunk=chunk):
                    pltpu.make_async_copy(
                        table_hbm.at[idx[b]], rows[b], sems[b]).wait()
                    pltpu.sync_copy(rows[b], out_hbm.at[pl.ds(chunk * CH, CH)])
                    nxt = chunk + 2 * nw

                    @pl.when(nxt < nchunks)
                    def _():
                        pltpu.sync_copy(nin_hbm.at[nxt], idx[b])
                        pltpu.async_copy(table_hbm.at[idx[b]], rows[b], sems[b])

            return carry

        lax.fori_loop(0, (cpw + 1) // 2, body, 0)

    return k(table, nin2d)


def _edge_body(eaT_ref, xs_ref, w1T_ref, b1c_ref, w2T_ref, b2T_ref, out_ref):
    eaT = eaT_ref[...]
    xT = xs_ref[:, :H].T
    hmidT = jnp.maximum(
        jnp.dot(w1T_ref[...], eaT, preferred_element_type=jnp.float32)
        + b1c_ref[:, 0:1], 0.0)
    pT = jnp.concatenate(
        [xT[j:j + 1, :] * hmidT for j in range(H)], axis=0)
    msgT = jnp.dot(w2T_ref[...], pT, preferred_element_type=jnp.float32) \
        + jnp.dot(b2T_ref[...], xT, preferred_element_type=jnp.float32)
    te = msgT.shape[1]
    out_ref[...] = jnp.concatenate(
        [msgT.T, jnp.zeros((te, DW - H), jnp.float32)], axis=1)


def _edge_msg(edge_attrT, x_src, w1T, b1c, w2T, b2T):
    TE = 1280
    e = x_src.shape[0]
    return pl.pallas_call(
        _edge_body,
        grid=(e // TE,),
        in_specs=[
            pl.BlockSpec((ED, TE), lambda i: (0, i)),
            pl.BlockSpec((TE, DW), lambda i: (i, 0)),
            pl.BlockSpec((H, ED), lambda i: (0, 0)),
            pl.BlockSpec((H, 8), lambda i: (0, 0)),
            pl.BlockSpec((H, H * H), lambda i: (0, 0)),
            pl.BlockSpec((H, H), lambda i: (0, 0)),
        ],
        out_specs=pl.BlockSpec((TE, DW), lambda i: (i, 0)),
        out_shape=jax.ShapeDtypeStruct((e, DW), jnp.float32),
    )(edge_attrT, x_src, w1T, b1c, w2T, b2T)


def _scatter_sc(msg, nout2d, zeros_np):
    mesh = plsc.VectorSubcoreMesh(core_axis_name="c", subcore_axis_name="s")
    half = nout2d.shape[0] // NCORE
    cpw = (half + NSUB - 1) // NSUB

    @functools.partial(
        pl.kernel, mesh=mesh,
        out_type=jax.ShapeDtypeStruct((NCORE, NP, H), jnp.float32),
        compiler_params=pltpu.CompilerParams(use_tc_tiling_on_sc=False),
        scratch_types=[
            pltpu.VMEM((CH,), jnp.int32),
            pltpu.VMEM((CH,), jnp.int32),
            pltpu.VMEM((CH, H), jnp.float32),
            pltpu.VMEM((CH, H), jnp.float32),
            pltpu.VMEM_SHARED((NP, H), jnp.float32),
            pltpu.SemaphoreType.DMA,
            pltpu.SemaphoreType.DMA,
        ],
    )
    def k(msg_hbm, nout_hbm, zeros_hbm, parts_hbm,
          idx0, idx1, rows0, rows1, acc_sh, sem0, sem1):
        idx = (idx0, idx1)
        rows = (rows0, rows1)
        sems = (sem0, sem1)
        c = lax.axis_index("c")
        s = lax.axis_index("s")
        pltpu.sync_copy(zeros_hbm.at[pl.ds(s * ZR, ZR)],
                        acc_sh.at[pl.ds(s * ZR, ZR)])

        for b in range(2):
            first = c * half + s + b * NSUB

            @pl.when(first < (c + 1) * half)
            def _(b=b, first=first):
                pltpu.sync_copy(nout_hbm.at[first], idx[b])
                pltpu.async_copy(
                    msg_hbm.at[pl.ds(first * CH, CH), pl.ds(0, H)],
                    rows[b], sems[b])

        plsc.subcore_barrier()

        def body(g, carry):
            for b in range(2):
                chunk = c * half + s + (g * 2 + b) * NSUB

                @pl.when(chunk < (c + 1) * half)
                def _(b=b, chunk=chunk):
                    pltpu.make_async_copy(
                        msg_hbm.at[pl.ds(chunk * CH, CH), pl.ds(0, H)],
                        rows[b], sems[b]).wait()
                    pltpu.sync_copy(rows[b], acc_sh.at[idx[b]], add=True)
                    nxt = chunk + 2 * NSUB

                    @pl.when(nxt < (c + 1) * half)
                    def _():
                        pltpu.sync_copy(nout_hbm.at[nxt], idx[b])
                        pltpu.async_copy(
                            msg_hbm.at[pl.ds(nxt * CH, CH), pl.ds(0, H)],
                            rows[b], sems[b])

            return carry

        lax.fori_loop(0, (cpw + 1) // 2, body, 0)
        plsc.subcore_barrier()
        pltpu.sync_copy(acc_sh.at[pl.ds(s * ZR, ZR)],
                        parts_hbm.at[c, pl.ds(s * ZR, ZR)])

    return k(msg, nout2d, zeros_np)


def _gru_body(up_ref, up2_ref, li_ref, giT_ref, ghT_ref, gb_ref, nf_ref):
    x = jnp.maximum(up_ref[0] + up_ref[1] + up2_ref[0] + up2_ref[1], 0.0)
    hx = li_ref[:, :H]
    gi = jnp.dot(x, giT_ref[...], preferred_element_type=jnp.float32) \
        + gb_ref[0:1, :]
    gh = jnp.dot(hx, ghT_ref[...], preferred_element_type=jnp.float32) \
        + gb_ref[1:2, :]
    r = jax.nn.sigmoid(gi[:, :H] + gh[:, :H])
    z = jax.nn.sigmoid(gi[:, H:2 * H] + gh[:, H:2 * H])
    nn_ = jnp.tanh(gi[:, 2 * H:] + r * gh[:, 2 * H:])
    nf_ref[...] = (1.0 - z) * nn_ + z * hx


def _gru(parts, parts2, li, giT, ghT, gb):
    TN = 2560
    return pl.pallas_call(
        _gru_body,
        grid=(NP // TN,),
        in_specs=[
            pl.BlockSpec((2, TN, H), lambda i: (0, i, 0)),
            pl.BlockSpec((2, TN, H), lambda i: (0, i, 0)),
            pl.BlockSpec((TN, DW), lambda i: (i, 0)),
            pl.BlockSpec((H, 3 * H), lambda i: (0, 0)),
            pl.BlockSpec((H, 3 * H), lambda i: (0, 0)),
            pl.BlockSpec((8, 3 * H), lambda i: (0, 0)),
        ],
        out_specs=pl.BlockSpec((TN, H), lambda i: (i, 0)),
        out_shape=jax.ShapeDtypeStruct((NP, H), jnp.float32),
    )(parts, parts2, li, giT, ghT, gb)


def _s2s_body(nf_ref, nfT_ref, n2gr_ref, liT_ref, lhT_ref, lb_ref, qs_ref):
    nf = nf_ref[...]
    nfT = nfT_ref[...]
    n2gr = n2gr_ref[0:1, :]
    ohT = (n2gr == lax.broadcasted_iota(jnp.int32, (B, NP), 0))
    ohf = ohT.astype(jnp.float32)

    h_l = jnp.zeros((B, H), jnp.float32)
    c_l = jnp.zeros((B, H), jnp.float32)
    q_star = jnp.zeros((B, 2 * H), jnp.float32)
    for _ in range(3):
        gates = jnp.dot(q_star, liT_ref[...], preferred_element_type=jnp.float32) \
            + jnp.dot(h_l, lhT_ref[...], preferred_element_type=jnp.float32) \
            + lb_ref[0:1, :]
        ig = jax.nn.sigmoid(gates[:, :H])
        fg = jax.nn.sigmoid(gates[:, H:2 * H])
        gg = jnp.tanh(gates[:, 2 * H:3 * H])
        og = jax.nn.sigmoid(gates[:, 3 * H:])
        c_l = fg * c_l + ig * gg
        h_l = og * jnp.tanh(c_l)
        q = h_l
        s = jnp.dot(q, nfT, preferred_element_type=jnp.float32)
        product = jnp.sum(ohf * s, axis=0, keepdims=True)
        pmax_b = jnp.max(jnp.where(ohT, product, -1e30), axis=1,
                         keepdims=True)
        pmax_n = jnp.sum(jnp.where(ohT, pmax_b, 0.0), axis=0,
                         keepdims=True)
        pexp = jnp.exp(product - pmax_n)
        psum_b = jnp.sum(ohf * pexp, axis=1, keepdims=True)
        psum_n = jnp.sum(jnp.where(ohT, psum_b, 0.0), axis=0,
                         keepdims=True)
        att = pexp / (psum_n + 1e-10)
        out = jnp.dot(ohf * att, nf, preferred_element_type=jnp.float32)
        q_star = jnp.concatenate([q, out], axis=1)
    qs_ref[...] = q_star


def _s2s(nf, nfT, n2gr, liT, lhT, lb):
    return pl.pallas_call(
        _s2s_body,
        out_shape=jax.ShapeDtypeStruct((B, 2 * H), jnp.float32),
        compiler_params=pltpu.CompilerParams(
            vmem_limit_bytes=64 * 1024 * 1024),
    )(nf, nfT, n2gr, liT, lhT, lb)


def kernel(unit_type, edge_index, edge_attr, node2graph, input, emb,
           mlp_w1, mlp_b1, mlp_w2, mlp_b2,
           gru_w_ih, gru_w_hh, gru_b_ih, gru_b_hh,
           lstm_w_ih, lstm_w_hh, lstm_b_ih, lstm_b_hh):
    ut_col = jnp.concatenate(
        [unit_type.astype(jnp.int32), jnp.zeros((NP - N,), jnp.int32)]
    ).reshape(NP, 1)
    n2g = jnp.concatenate(
        [node2graph.astype(jnp.int32), jnp.full((NP - N,), B, jnp.int32)])
    n2gr = jnp.broadcast_to(n2g.reshape(1, NP), (8, NP))

    nin2d = edge_index[0].astype(jnp.int32).reshape(NCHUNKS, CH)
    nout2d = edge_index[1].astype(jnp.int32).reshape(NCHUNKS, CH)

    w2T = mlp_w2.reshape(H, H, H).transpose(1, 2, 0).reshape(H, H * H)
    b2T = mlp_b2.reshape(H, H)
    w1T = mlp_w1.T
    b1c = jnp.broadcast_to(mlp_b1.reshape(H, 1), (H, 8))
    eaT = edge_attr.T

    giT = gru_w_ih.T
    ghT = gru_w_hh.T
    gb = jnp.concatenate(
        [gru_b_ih.reshape(1, 3 * H), gru_b_hh.reshape(1, 3 * H)], axis=0)
    gb = jnp.concatenate([gb, jnp.zeros((6, 3 * H), jnp.float32)], axis=0)
    liT = lstm_w_ih.T
    lhT = lstm_w_hh.T
    lb = jnp.broadcast_to(
        (lstm_b_ih + lstm_b_hh).reshape(1, 4 * H), (8, 4 * H))

    emb128 = jnp.concatenate(
        [emb, jnp.zeros((NUM_UNIT, DW - H), jnp.float32)], axis=1)
    layer_input = _embed(ut_col, emb128)
    E0 = 79360
    C0 = E0 // CH
    zeros_np = jnp.zeros((NP, H), jnp.float32)

    x0 = _gather_sc(layer_input, nin2d[:C0])
    x1 = _gather_sc(layer_input, nin2d[C0:])
    msg0 = _edge_msg(eaT[:, :E0], x0, w1T, b1c, w2T, b2T)
    msg1 = _edge_msg(eaT[:, E0:], x1, w1T, b1c, w2T, b2T)
    parts0 = _scatter_sc(msg0, nout2d[:C0], zeros_np)
    parts1 = _scatter_sc(msg1, nout2d[C0:], zeros_np)
    nf = _gru(parts0, parts1, layer_input, giT, ghT, gb)
    q_star = _s2s(nf, nf.T, n2gr, liT, lhT, lb)
    return (q_star, nf[:N])
